# Initial kernel scaffold; baseline (speedup 1.0000x reference)
import jax
import jax.numpy as jnp
from jax.experimental import pallas as pl


def kernel(x, edge_index, hyperedge_index, W1, b1, Th1, bh1, A1_W, A1_v, W2, b2, Th2, bh2, A2_W, A2_v):
    raise NotImplementedError("write your pallas kernel here")


# speedup vs baseline: 12.5559x; 12.5559x over previous
import functools

import jax
import jax.numpy as jnp
from jax import lax
from jax.experimental import pallas as pl
from jax.experimental.pallas import tpu as pltpu
from jax.experimental.pallas import tpu_sc as plsc

N = 10000
E = 320000
NHE = 10000
F_IN = 128
DIM = 64
C = 40

NC, NS = 2, 16
NW = NC * NS
EPW = E // NW
CH = 125
NCH = EPW // CH
D0 = 624
D15 = N - 15 * D0

_mesh = lambda: plsc.VectorSubcoreMesh(
    core_axis_name="c", subcore_axis_name="s", num_cores=NC, num_subcores=NS)


HW = 16

@functools.partial(
    pl.kernel,
    out_type=jax.ShapeDtypeStruct((NC, 3, N, HW), jnp.float32),
    mesh=_mesh(),
    compiler_params=pltpu.CompilerParams(use_tc_tiling_on_sc=False),
    scratch_types=[
        pltpu.VMEM_SHARED((N, HW), jnp.float32),
        pltpu.VMEM_SHARED((N, HW), jnp.float32),
        pltpu.VMEM_SHARED((N, HW), jnp.float32),
        pltpu.VMEM((NCH, CH), jnp.int32),
        pltpu.VMEM((CH, HW), jnp.float32),
    ],
)
def _deg_kernel(dst_h, hn_h, he_h, ones_h, zeros_h, out_h,
                hd, hn, he, idx, ones_v):
    c = lax.axis_index("c")
    s = lax.axis_index("s")
    w = c * NS + s

    @pl.when(s == 0)
    def _zero():
        pltpu.sync_copy(zeros_h, hd)
        pltpu.sync_copy(zeros_h, hn)
        pltpu.sync_copy(zeros_h, he)

    pltpu.sync_copy(ones_h, ones_v)
    plsc.subcore_barrier()

    for src_h, hist in ((dst_h, hd), (hn_h, hn), (he_h, he)):
        pltpu.sync_copy(src_h.at[w], idx)

        def chunk(ci, carry):
            pltpu.sync_copy(ones_v, hist.at[idx.at[ci]], add=True)
            return carry

        lax.fori_loop(0, NCH, chunk, 0)

    plsc.subcore_barrier()

    @pl.when(s == 0)
    def _dump():
        pltpu.sync_copy(hd, out_h.at[c, 0])
        pltpu.sync_copy(hn, out_h.at[c, 1])
        pltpu.sync_copy(he, out_h.at[c, 2])


def _zero_acc(zeros_h, acc, s):
    @pl.when(s < 15)
    def _():
        sl = pl.ds(pl.multiple_of(s * D0, 8), D0)
        pltpu.sync_copy(zeros_h.at[pl.ds(0, D0)], acc.at[sl])

    @pl.when(s == 15)
    def _():
        pltpu.sync_copy(zeros_h, acc.at[pl.ds(15 * D0, D15)])


def _dump_acc(acc, out, c, s):
    @pl.when(s < 15)
    def _():
        sl = pl.ds(pl.multiple_of(s * D0, 8), D0)
        pltpu.sync_copy(acc.at[sl], out.at[c, sl])

    @pl.when(s == 15)
    def _():
        sl = pl.ds(15 * D0, D15)
        pltpu.sync_copy(acc.at[sl], out.at[c, sl])


def _pass_body(tab, gi, si, acc, w, idxg, idxs, rows):
    pltpu.sync_copy(gi.at[w], idxg)
    pltpu.sync_copy(si.at[w], idxs)

    def chunk(ci, carry):
        pltpu.sync_copy(tab.at[idxg.at[ci]], rows)
        pltpu.sync_copy(rows, acc.at[idxs.at[ci]], add=True)
        return carry

    lax.fori_loop(0, NCH, chunk, 0)


@functools.partial(
    pl.kernel,
    out_type=(jax.ShapeDtypeStruct((NC, N, DIM), jnp.float32),
              jax.ShapeDtypeStruct((NC, NHE, DIM), jnp.float32)),
    mesh=_mesh(),
    compiler_params=pltpu.CompilerParams(use_tc_tiling_on_sc=False),
    scratch_types=[
        pltpu.VMEM_SHARED((N, DIM), jnp.float32),
        pltpu.VMEM_SHARED((NHE, DIM), jnp.float32),
        pltpu.VMEM((NCH, CH), jnp.int32),
        pltpu.VMEM((NCH, CH), jnp.int32),
        pltpu.VMEM((CH, DIM), jnp.float32),
    ],
)
def _pass2_kernel(tab1, g1, s1, tab2, g2, s2, zeros_h, out1, out2,
                  acc1, acc2, idxg, idxs, rows):
    c = lax.axis_index("c")
    s = lax.axis_index("s")
    w = c * NS + s
    _zero_acc(zeros_h, acc1, s)
    _zero_acc(zeros_h, acc2, s)
    plsc.subcore_barrier()
    _pass_body(tab1, g1, s1, acc1, w, idxg, idxs, rows)
    _pass_body(tab2, g2, s2, acc2, w, idxg, idxs, rows)
    plsc.subcore_barrier()
    _dump_acc(acc1, out1, c, s)
    _dump_acc(acc2, out2, c, s)


@functools.partial(
    pl.kernel,
    out_type=jax.ShapeDtypeStruct((NC, N, DIM), jnp.float32),
    mesh=_mesh(),
    compiler_params=pltpu.CompilerParams(use_tc_tiling_on_sc=False),
    scratch_types=[
        pltpu.VMEM_SHARED((N, DIM), jnp.float32),
        pltpu.VMEM((NCH, CH), jnp.int32),
        pltpu.VMEM((NCH, CH), jnp.int32),
        pltpu.VMEM((CH, DIM), jnp.float32),
    ],
)
def _pass1_kernel(tab1, g1, s1, zeros_h, out1, acc1, idxg, idxs, rows):
    c = lax.axis_index("c")
    s = lax.axis_index("s")
    w = c * NS + s
    _zero_acc(zeros_h, acc1, s)
    plsc.subcore_barrier()
    _pass_body(tab1, g1, s1, acc1, w, idxg, idxs, rows)
    plsc.subcore_barrier()
    _dump_acc(acc1, out1, c, s)


BN = 1000
GRID = N // BN

def _col_spec():
    return pl.BlockSpec((BN, 1), lambda i: (i, 0))

def _row_spec(d=DIM):
    return pl.BlockSpec((BN, d), lambda i: (i, 0))

def _full_spec(a, b):
    return pl.BlockSpec((a, b), lambda i: (0, 0))


def _tc1_body(x_r, w1_r, th1_r, cd0_r, cd1_r, hp_r, hh_r):
    x = x_r[...]
    dinv = lax.rsqrt(cd0_r[...] + cd1_r[...] + 1.0)
    hp_r[...] = jnp.dot(x, w1_r[...], preferred_element_type=jnp.float32) * dinv
    hh_r[...] = jnp.dot(x, th1_r[...], preferred_element_type=jnp.float32)


def _tc1(x, W1, Th1, cd0, cd1):
    return pl.pallas_call(
        _tc1_body,
        grid=(GRID,),
        in_specs=[_row_spec(F_IN), _full_spec(F_IN, DIM), _full_spec(F_IN, DIM),
                  _col_spec(), _col_spec()],
        out_specs=[_row_spec(), _row_spec()],
        out_shape=[jax.ShapeDtypeStruct((N, DIM), jnp.float32),
                   jax.ShapeDtypeStruct((N, DIM), jnp.float32)],
    )(x, W1, Th1, cd0, cd1)


def _scale_body(p0_r, p1_r, c0_r, c1_r, o_r):
    cnt = c0_r[...] + c1_r[...]
    inv = jnp.where(cnt > 0, 1.0 / cnt, 0.0)
    o_r[...] = (p0_r[...] + p1_r[...]) * inv


def _scale(p0, p1, c0, c1):
    return pl.pallas_call(
        _scale_body,
        grid=(GRID,),
        in_specs=[_row_spec(), _row_spec(), _col_spec(), _col_spec()],
        out_specs=pl.BlockSpec((BN, DIM), lambda i: (i, 0)),
        out_shape=jax.ShapeDtypeStruct((NHE, DIM), jnp.float32),
    )(p0, p1, c0, c1)


def _attn(zg, zh, AW, Av):
    wg = jnp.dot(jnp.tanh(jnp.dot(zg, AW, preferred_element_type=jnp.float32)),
                 Av, preferred_element_type=jnp.float32)
    wh = jnp.dot(jnp.tanh(jnp.dot(zh, AW, preferred_element_type=jnp.float32)),
                 Av, preferred_element_type=jnp.float32)
    m = jnp.maximum(wg, wh)
    eg = jnp.exp(wg - m)
    eh = jnp.exp(wh - m)
    tot = eg + eh
    return (eg / tot) * zg + (eh / tot) * zh


def _tc3_body(ag0_r, ag1_r, hp_r, cd0_r, cd1_r, ho0_r, ho1_r, cn0_r, cn1_r,
              b1_r, bh1_r, a1w_r, a1v_r, w2_r, th2_r, hp2_r, hh2_r):
    dinv = lax.rsqrt(cd0_r[...] + cd1_r[...] + 1.0)
    xg = jax.nn.relu(dinv * (ag0_r[...] + ag1_r[...] + hp_r[...]) + b1_r[...])
    cn = cn0_r[...] + cn1_r[...]
    Dinv = jnp.where(cn > 0, 1.0 / cn, 0.0)
    xh = jax.nn.relu(Dinv * (ho0_r[...] + ho1_r[...]) + bh1_r[...])
    xc = _attn(xg, xh, a1w_r[...], a1v_r[...])
    hp2_r[...] = dinv * jnp.dot(xc, w2_r[...], preferred_element_type=jnp.float32)
    hh2_r[...] = jnp.dot(xh, th2_r[...], preferred_element_type=jnp.float32)


def _tc3(ag0, ag1, hp, cd0, cd1, ho0, ho1, cn0, cn1, b1r, bh1r, A1W, A1v, W2p, Th2p):
    return pl.pallas_call(
        _tc3_body,
        grid=(GRID,),
        in_specs=[_row_spec(), _row_spec(), _row_spec(), _col_spec(), _col_spec(),
                  _row_spec(), _row_spec(), _col_spec(), _col_spec(),
                  _full_spec(1, DIM), _full_spec(1, DIM),
                  _full_spec(DIM, DIM), _full_spec(DIM, 1),
                  _full_spec(DIM, DIM), _full_spec(DIM, DIM)],
        out_specs=[_row_spec(), _row_spec()],
        out_shape=[jax.ShapeDtypeStruct((N, DIM), jnp.float32),
                   jax.ShapeDtypeStruct((N, DIM), jnp.float32)],
    )(ag0, ag1, hp, cd0, cd1, ho0, ho1, cn0, cn1, b1r, bh1r, A1W, A1v, W2p, Th2p)


def _tc5_body(ag0_r, ag1_r, hp2_r, cd0_r, cd1_r, ho0_r, ho1_r, cn0_r, cn1_r,
              b2_r, bh2_r, a2w_r, a2v_r, o_r):
    dinv = lax.rsqrt(cd0_r[...] + cd1_r[...] + 1.0)
    xg = dinv * (ag0_r[...] + ag1_r[...] + hp2_r[...]) + b2_r[...]
    cn = cn0_r[...] + cn1_r[...]
    Dinv = jnp.where(cn > 0, 1.0 / cn, 0.0)
    xh = Dinv * (ho0_r[...] + ho1_r[...]) + bh2_r[...]
    xo = _attn(xg, xh, a2w_r[...], a2v_r[...])[:, :C]
    m = jnp.max(xo, axis=1, keepdims=True)
    e = jnp.exp(xo - m)
    o_r[...] = xo - m - jnp.log(jnp.sum(e, axis=1, keepdims=True))


def _tc5(ag0, ag1, hp2, cd0, cd1, ho0, ho1, cn0, cn1, b2r, bh2r, A2W, A2v):
    return pl.pallas_call(
        _tc5_body,
        grid=(GRID,),
        in_specs=[_row_spec(), _row_spec(), _row_spec(), _col_spec(), _col_spec(),
                  _row_spec(), _row_spec(), _col_spec(), _col_spec(),
                  _full_spec(1, DIM), _full_spec(1, DIM),
                  _full_spec(DIM, DIM), _full_spec(DIM, 1)],
        out_specs=pl.BlockSpec((BN, C), lambda i: (i, 0)),
        out_shape=jax.ShapeDtypeStruct((N, C), jnp.float32),
    )(ag0, ag1, hp2, cd0, cd1, ho0, ho1, cn0, cn1, b2r, bh2r, A2W, A2v)


def kernel(x, edge_index, hyperedge_index, W1, b1, Th1, bh1, A1_W, A1_v,
           W2, b2, Th2, bh2, A2_W, A2_v):
    f32 = jnp.float32
    src = edge_index[0].reshape(NW, NCH, CH)
    dst = edge_index[1].reshape(NW, NCH, CH)
    hn = hyperedge_index[0].reshape(NW, NCH, CH)
    he = hyperedge_index[1].reshape(NW, NCH, CH)

    ones_col = jnp.ones((CH, HW), f32)
    zeros_col = jnp.zeros((N, HW), f32)
    zeros_blk = jnp.zeros((D15, DIM), f32)

    W2p = jnp.zeros((DIM, DIM), f32).at[:, :C].set(W2)
    Th2p = jnp.zeros((DIM, DIM), f32).at[:, :C].set(Th2)
    b2r = jnp.zeros((1, DIM), f32).at[0, :C].set(b2)
    bh2r = jnp.zeros((1, DIM), f32).at[0, :C].set(bh2)
    A2Wp = jnp.zeros((DIM, DIM), f32).at[:C, :C].set(A2_W)
    A2vp = jnp.zeros((DIM, 1), f32).at[:C, 0].set(A2_v)
    b1r = b1.reshape(1, DIM)
    bh1r = bh1.reshape(1, DIM)
    A1vr = A1_v.reshape(DIM, 1)

    cnt = _deg_kernel(dst, hn, he, ones_col, zeros_col)
    cd0, cd1 = cnt[0, 0, :, :1], cnt[1, 0, :, :1]
    cn0, cn1 = cnt[0, 1, :, :1], cnt[1, 1, :, :1]
    cb0, cb1 = cnt[0, 2, :, :1], cnt[1, 2, :, :1]

    hp, hh1 = _tc1(x, W1, Th1, cd0, cd1)

    aggp, hep = _pass2_kernel(hp, src, dst, hh1, hn, he, zeros_blk)
    hes = _scale(hep[0], hep[1], cb0, cb1)
    hop = _pass1_kernel(hes, he, hn, zeros_blk)

    hp2, hh2 = _tc3(aggp[0], aggp[1], hp, cd0, cd1, hop[0], hop[1], cn0, cn1,
                    b1r, bh1r, A1_W, A1vr, W2p, Th2p)

    agg2p, he2p = _pass2_kernel(hp2, src, dst, hh2, hn, he, zeros_blk)
    he2s = _scale(he2p[0], he2p[1], cb0, cb1)
    ho2p = _pass1_kernel(he2s, he, hn, zeros_blk)

    return _tc5(agg2p[0], agg2p[1], hp2, cd0, cd1, ho2p[0], ho2p[1], cn0, cn1,
                b2r, bh2r, A2Wp, A2vp)
